# baseline (speedup 1.0000x reference)
import jax
import jax.numpy as jnp
from jax.experimental import pallas as pl
from jax.experimental.pallas import tpu as pltpu

_B, _T, _N, _D = 4, 32, 257, 1024


def _copy_body(img_ref, out_ref, sem):
    pltpu.make_async_copy(img_ref.at[:, 0], out_ref, sem).start()
    pltpu.make_async_copy(img_ref.at[:, 0], out_ref, sem).wait()


_cls_slice = pl.pallas_call(
    _copy_body,
    in_specs=[pl.BlockSpec(memory_space=pl.ANY)],
    out_specs=pl.BlockSpec(memory_space=pl.ANY),
    out_shape=jax.ShapeDtypeStruct((_B, _T, _D), jnp.float32),
    scratch_shapes=[pltpu.SemaphoreType.DMA],
)


def kernel(image):
    return _cls_slice(jnp.transpose(image, (0, 2, 1, 3)))


# speedup vs baseline: 8.9163x; 8.9163x over previous
import jax
import jax.numpy as jnp
from jax.experimental import pallas as pl
from jax.experimental.pallas import tpu as pltpu

_B, _T, _N, _D = 4, 32, 257, 1024


def _copy_body(img_ref, out_ref):
    out_ref[...] = img_ref[:, 0]


_cls_slice = pl.pallas_call(
    _copy_body,
    grid=(1,),
    in_specs=[
        pl.BlockSpec((_B, 1, _T, _D), lambda i: (0, 0, 0, 0)),
    ],
    out_specs=pl.BlockSpec((_B, _T, _D), lambda i: (0, 0, 0)),
    out_shape=jax.ShapeDtypeStruct((_B, _T, _D), jnp.float32),
)


def kernel(image):
    return _cls_slice(jnp.transpose(image, (0, 2, 1, 3)))


# speedup vs baseline: 9.5764x; 1.0740x over previous
import jax
import jax.numpy as jnp
from jax.experimental import pallas as pl
from jax.experimental.pallas import tpu as pltpu

_B, _T, _N, _D = 4, 32, 257, 1024


def _copy_body(img_ref, out_ref, buf, sem_in, sem_out):
    for b in range(_B):
        pltpu.make_async_copy(img_ref.at[b, 0], buf.at[b], sem_in).start()
    for b in range(_B):
        pltpu.make_async_copy(img_ref.at[b, 0], buf.at[b], sem_in).wait()
        pltpu.make_async_copy(buf.at[b], out_ref.at[b], sem_out).start()
    for b in range(_B):
        pltpu.make_async_copy(buf.at[b], out_ref.at[b], sem_out).wait()


_cls_slice = pl.pallas_call(
    _copy_body,
    in_specs=[pl.BlockSpec(memory_space=pl.ANY)],
    out_specs=pl.BlockSpec(memory_space=pl.ANY),
    out_shape=jax.ShapeDtypeStruct((_B, _T, _D), jnp.float32),
    scratch_shapes=[
        pltpu.VMEM((_B, _T, _D), jnp.float32),
        pltpu.SemaphoreType.DMA,
        pltpu.SemaphoreType.DMA,
    ],
)


def kernel(image):
    return _cls_slice(jnp.transpose(image, (0, 2, 1, 3)))


# speedup vs baseline: 9.6381x; 1.0064x over previous
import jax
import jax.numpy as jnp
from jax.experimental import pallas as pl
from jax.experimental.pallas import tpu as pltpu

_B, _T, _N, _D = 4, 32, 257, 1024


_CH = 2
_TC = _T // _CH


def _chunks(img_ref, out_ref, buf):
    for b in range(_B):
        for c in range(_CH):
            t0 = c * _TC
            yield (
                img_ref.at[b, 0, pl.ds(t0, _TC)],
                buf.at[b, pl.ds(t0, _TC)],
                out_ref.at[b, pl.ds(t0, _TC)],
            )


def _copy_body(img_ref, out_ref, buf, sem_in, sem_out):
    for src, mid, _ in _chunks(img_ref, out_ref, buf):
        pltpu.make_async_copy(src, mid, sem_in).start()
    for src, mid, dst in _chunks(img_ref, out_ref, buf):
        pltpu.make_async_copy(src, mid, sem_in).wait()
        pltpu.make_async_copy(mid, dst, sem_out).start()
    for _, mid, dst in _chunks(img_ref, out_ref, buf):
        pltpu.make_async_copy(mid, dst, sem_out).wait()


_cls_slice = pl.pallas_call(
    _copy_body,
    in_specs=[pl.BlockSpec(memory_space=pl.ANY)],
    out_specs=pl.BlockSpec(memory_space=pl.ANY),
    out_shape=jax.ShapeDtypeStruct((_B, _T, _D), jnp.float32),
    scratch_shapes=[
        pltpu.VMEM((_B, _T, _D), jnp.float32),
        pltpu.SemaphoreType.DMA,
        pltpu.SemaphoreType.DMA,
    ],
)


def kernel(image):
    return _cls_slice(jnp.transpose(image, (0, 2, 1, 3)))


# speedup vs baseline: 10.0069x; 1.0383x over previous
import jax
import jax.numpy as jnp
from jax.experimental import pallas as pl
from jax.experimental.pallas import tpu as pltpu

_B, _T, _N, _D = 4, 32, 257, 1024


_CH = 4
_TC = _T // _CH


def _chunks(img_ref, out_ref, buf):
    for b in range(_B):
        for c in range(_CH):
            t0 = c * _TC
            yield (
                img_ref.at[b, 0, pl.ds(t0, _TC)],
                buf.at[b, pl.ds(t0, _TC)],
                out_ref.at[b, pl.ds(t0, _TC)],
            )


def _copy_body(img_ref, out_ref, buf, sem_in, sem_out):
    for src, mid, _ in _chunks(img_ref, out_ref, buf):
        pltpu.make_async_copy(src, mid, sem_in).start()
    for src, mid, dst in _chunks(img_ref, out_ref, buf):
        pltpu.make_async_copy(src, mid, sem_in).wait()
        pltpu.make_async_copy(mid, dst, sem_out).start()
    for _, mid, dst in _chunks(img_ref, out_ref, buf):
        pltpu.make_async_copy(mid, dst, sem_out).wait()


_cls_slice = pl.pallas_call(
    _copy_body,
    in_specs=[pl.BlockSpec(memory_space=pl.ANY)],
    out_specs=pl.BlockSpec(memory_space=pl.ANY),
    out_shape=jax.ShapeDtypeStruct((_B, _T, _D), jnp.float32),
    scratch_shapes=[
        pltpu.VMEM((_B, _T, _D), jnp.float32),
        pltpu.SemaphoreType.DMA,
        pltpu.SemaphoreType.DMA,
    ],
)


def kernel(image):
    return _cls_slice(jnp.transpose(image, (0, 2, 1, 3)))
